# baseline (speedup 1.0000x reference)
import functools

import jax
import jax.numpy as jnp
from jax import lax
from jax.experimental import pallas as pl
from jax.experimental.pallas import tpu as pltpu
from jax.experimental.pallas import tpu_sc as plsc

N_NODES = 10000
N_EDGES = 320000
D = 128
EPS = 1e-5

NC = 2
NS = 16
NW = NC * NS
CHUNK = 128
CH_PER_TILE = 80
IDXB = 8
E_PAD = NW * CH_PER_TILE * CHUNK
NPAD = 10240
RPS = NPAD // NS
ZCH = RPS // CHUNK


def _sc_body(x_hbm, rowc_hbm, colc_hbm, iota_hbm, agg_hbm, deg_hbm,
             agg_sh, rowi_v, coli_v, idxz_v, rows_a, rows_b,
             gsem_a, gsem_b, ssem_a, ssem_b, dsem):
    cid = lax.axis_index("c")
    sid = lax.axis_index("s")
    wid = cid * NS + sid
    cb = wid * CH_PER_TILE
    ob = cid * NPAD + sid * RPS

    z16 = jnp.zeros((16,), jnp.float32)
    o16 = jnp.ones((16,), jnp.float32)

    def fill_rows(v16):
        def init_row(r, _):
            for l in range(D // 16):
                rows_a[r, pl.ds(16 * l, 16)] = v16
            return 0

        lax.fori_loop(0, CHUNK, init_row, 0)

    def zero_acc():
        def zfill(k, _):
            pltpu.sync_copy(rows_a, agg_sh.at[idxz_v.at[k]])
            return 0

        lax.fori_loop(0, ZCH, zfill, 0)

    def copy_out(dst_hbm):
        def out_chunk(k, _):
            pltpu.async_copy(agg_sh.at[idxz_v.at[k]], rows_a, gsem_a).wait()
            pltpu.sync_copy(rows_a, dst_hbm.at[pl.ds(ob + k * CHUNK, CHUNK)])
            return 0

        lax.fori_loop(0, ZCH, out_chunk, 0)

    pltpu.sync_copy(iota_hbm.at[sid], idxz_v)
    pltpu.sync_copy(rowc_hbm.at[pl.ds(cb, CH_PER_TILE)], rowi_v)

    fill_rows(z16)
    zero_acc()
    plsc.subcore_barrier()

    HALF = CH_PER_TILE // 2
    NPAIR = HALF // 2

    for hp in range(2):
        hc = hp * HALF
        pltpu.sync_copy(colc_hbm.at[pl.ds(cb + hc, HALF)], coli_v)
        pltpu.async_copy(x_hbm.at[coli_v.at[0]], rows_a, gsem_a)
        pltpu.async_copy(x_hbm.at[coli_v.at[1]], rows_b, gsem_b)

        def pair(t, _):
            pltpu.make_async_copy(x_hbm.at[coli_v.at[2 * t]], rows_a,
                                  gsem_a).wait()
            pltpu.async_copy(rows_a, agg_sh.at[rowi_v.at[hc + 2 * t]],
                             ssem_a, add=True)
            pltpu.make_async_copy(x_hbm.at[coli_v.at[2 * t + 1]], rows_b,
                                  gsem_b).wait()
            pltpu.async_copy(rows_b, agg_sh.at[rowi_v.at[hc + 2 * t + 1]],
                             ssem_b, add=True)

            pltpu.make_async_copy(rows_a, agg_sh.at[rowi_v.at[hc + 2 * t]],
                                  ssem_a).wait()

            @pl.when(t < NPAIR - 1)
            def _():
                pltpu.async_copy(x_hbm.at[coli_v.at[2 * t + 2]], rows_a,
                                 gsem_a)

            pltpu.make_async_copy(rows_b,
                                  agg_sh.at[rowi_v.at[hc + 2 * t + 1]],
                                  ssem_b).wait()

            @pl.when(t < NPAIR - 1)
            def _():
                pltpu.async_copy(x_hbm.at[coli_v.at[2 * t + 3]], rows_b,
                                 gsem_b)

            return 0

        lax.fori_loop(0, NPAIR, pair, 0)

    plsc.subcore_barrier()
    copy_out(agg_hbm)
    plsc.subcore_barrier()

    fill_rows(o16)

    def batch_deg(bi, _):
        for j in range(IDXB):
            pltpu.async_copy(rows_a, agg_sh.at[rowi_v.at[bi * IDXB + j]],
                             dsem, add=True)
        for j in range(IDXB):
            pltpu.make_async_copy(rows_a,
                                  agg_sh.at[rowi_v.at[bi * IDXB + j]],
                                  dsem).wait()
        return 0

    lax.fori_loop(0, CH_PER_TILE // IDXB, batch_deg, 0)
    plsc.subcore_barrier()
    copy_out(deg_hbm)


@functools.cache
def _sc_scatter_kernel():
    return pl.kernel(
        _sc_body,
        out_type=(
            jax.ShapeDtypeStruct((NC * NPAD, D), jnp.float32),
            jax.ShapeDtypeStruct((NC * NPAD, D), jnp.float32),
        ),
        mesh=plsc.VectorSubcoreMesh(core_axis_name="c", subcore_axis_name="s",
                                    num_cores=NC, num_subcores=NS),
        scratch_types=[
            pltpu.MemorySpace.VMEM_SHARED((NPAD, D), jnp.float32),
            pltpu.VMEM((CH_PER_TILE, CHUNK), jnp.int32),
            pltpu.VMEM((CH_PER_TILE // 2, CHUNK), jnp.int32),
            pltpu.VMEM((ZCH, CHUNK), jnp.int32),
            pltpu.VMEM((CHUNK, D), jnp.float32),
            pltpu.VMEM((CHUNK, D), jnp.float32),
            pltpu.SemaphoreType.DMA,
            pltpu.SemaphoreType.DMA,
            pltpu.SemaphoreType.DMA,
            pltpu.SemaphoreType.DMA,
            pltpu.SemaphoreType.DMA,
        ],
    )


ROWB = 1024
NBLK = NPAD // ROWB


def _lin_body(x_ref, agg_ref, deg_ref, w_ref, b_ref,
              h_ref, s_ref, q_ref, sacc, qacc):
    i = pl.program_id(0)
    agg = agg_ref[0] + agg_ref[1]
    valid = (lax.broadcasted_iota(jnp.int32, (ROWB, 1), 0) + i * ROWB
             < N_NODES).astype(jnp.float32)
    deg = (deg_ref[0, :, 0:1] - agg_ref[0, :, 0:1]
           + deg_ref[1, :, 0:1] - agg_ref[1, :, 0:1])
    scale = 1.0 / jnp.maximum(deg, 1.0)
    hin = x_ref[...] + agg * scale
    h = lax.dot_general(hin, w_ref[...], (((1,), (1,)), ((), ())),
                        preferred_element_type=jnp.float32) + b_ref[...]
    h_ref[...] = h

    @pl.when(i == 0)
    def _():
        sacc[...] = jnp.zeros_like(sacc)
        qacc[...] = jnp.zeros_like(qacc)

    hm = h * valid
    sacc[...] += jnp.sum(hm, axis=0, keepdims=True)
    qacc[...] += jnp.sum(hm * hm, axis=0, keepdims=True)

    @pl.when(i == NBLK - 1)
    def _():
        s_ref[...] = sacc[...]
        q_ref[...] = qacc[...]


def _norm_body(h_ref, s_ref, q_ref, g_ref, be_ref, o_ref):
    inv_n = 1.0 / N_NODES
    mean = s_ref[...] * inv_n
    var = q_ref[...] * inv_n - mean * mean
    rstd = lax.rsqrt(var + EPS)
    o_ref[...] = jnp.maximum(
        (h_ref[...] - mean) * rstd * g_ref[...] + be_ref[...], 0.0)


def kernel(x, edge_index, W, b, gamma, beta):
    row = edge_index[0]
    col = edge_index[1]
    pad = E_PAD - N_EDGES
    row_p = jnp.concatenate([row, jnp.full((pad,), N_NODES, jnp.int32)])
    col_p = jnp.concatenate([col, jnp.zeros((pad,), jnp.int32)])
    rowc = row_p.reshape(NW * CH_PER_TILE, CHUNK)
    colc = col_p.reshape(NW * CH_PER_TILE, CHUNK)

    iota = jnp.arange(NPAD, dtype=jnp.int32).reshape(NS, ZCH, CHUNK)

    agg_flat, deg_flat = _sc_scatter_kernel()(x, rowc, colc, iota)
    agg3 = agg_flat.reshape(NC, NPAD, D)
    deg3 = deg_flat.reshape(NC, NPAD, D)
    x_p = jnp.concatenate(
        [x, jnp.zeros((NPAD - N_NODES, D), jnp.float32)], axis=0)

    b2 = b.reshape(1, D)
    g2 = gamma.reshape(1, D)
    be2 = beta.reshape(1, D)

    h, ssum, ssq = pl.pallas_call(
        _lin_body,
        grid=(NBLK,),
        in_specs=[
            pl.BlockSpec((ROWB, D), lambda i: (i, 0)),
            pl.BlockSpec((NC, ROWB, D), lambda i: (0, i, 0)),
            pl.BlockSpec((NC, ROWB, D), lambda i: (0, i, 0)),
            pl.BlockSpec((D, D), lambda i: (0, 0)),
            pl.BlockSpec((1, D), lambda i: (0, 0)),
        ],
        out_specs=[
            pl.BlockSpec((ROWB, D), lambda i: (i, 0)),
            pl.BlockSpec((1, D), lambda i: (0, 0)),
            pl.BlockSpec((1, D), lambda i: (0, 0)),
        ],
        out_shape=[
            jax.ShapeDtypeStruct((NPAD, D), jnp.float32),
            jax.ShapeDtypeStruct((1, D), jnp.float32),
            jax.ShapeDtypeStruct((1, D), jnp.float32),
        ],
        scratch_shapes=[
            pltpu.VMEM((1, D), jnp.float32),
            pltpu.VMEM((1, D), jnp.float32),
        ],
    )(x_p, agg3, deg3, W, b2)

    out = pl.pallas_call(
        _norm_body,
        grid=(NBLK,),
        in_specs=[
            pl.BlockSpec((ROWB, D), lambda i: (i, 0)),
            pl.BlockSpec((1, D), lambda i: (0, 0)),
            pl.BlockSpec((1, D), lambda i: (0, 0)),
            pl.BlockSpec((1, D), lambda i: (0, 0)),
            pl.BlockSpec((1, D), lambda i: (0, 0)),
        ],
        out_specs=pl.BlockSpec((ROWB, D), lambda i: (i, 0)),
        out_shape=jax.ShapeDtypeStruct((NPAD, D), jnp.float32),
    )(h, ssum, ssq, g2, be2)

    return out[:N_NODES]


# speedup vs baseline: 1.0201x; 1.0201x over previous
import functools

import jax
import jax.numpy as jnp
from jax import lax
from jax.experimental import pallas as pl
from jax.experimental.pallas import tpu as pltpu
from jax.experimental.pallas import tpu_sc as plsc

N_NODES = 10000
N_EDGES = 320000
D = 128
EPS = 1e-5

NC = 2
NS = 16
NW = NC * NS
CHUNK = 128
CH_PER_TILE = 80
IDXB = 8
E_PAD = NW * CH_PER_TILE * CHUNK
NPAD = 10240
RPS = NPAD // NS
ZCH = RPS // CHUNK


def _sc_body(x_hbm, rowc_hbm, colc_hbm, iota_hbm, agg_hbm, deg_hbm,
             agg_sh, rowi_v, coli_v, idxz_v, rows_a, rows_b,
             gsem_a, gsem_b, dsem):
    cid = lax.axis_index("c")
    sid = lax.axis_index("s")
    wid = cid * NS + sid
    cb = wid * CH_PER_TILE
    ob = cid * NPAD + sid * RPS

    z16 = jnp.zeros((16,), jnp.float32)
    o16 = jnp.ones((16,), jnp.float32)

    def fill_rows(v16):
        def init_row(r, _):
            for l in range(D // 16):
                rows_a[r, pl.ds(16 * l, 16)] = v16
            return 0

        lax.fori_loop(0, CHUNK, init_row, 0)

    def zero_acc():
        def zfill(k, _):
            pltpu.sync_copy(rows_a, agg_sh.at[idxz_v.at[k]])
            return 0

        lax.fori_loop(0, ZCH, zfill, 0)

    def copy_out(dst_hbm):
        def out_chunk(k, _):
            pltpu.async_copy(agg_sh.at[idxz_v.at[k]], rows_a, gsem_a).wait()
            pltpu.sync_copy(rows_a, dst_hbm.at[pl.ds(ob + k * CHUNK, CHUNK)])
            return 0

        lax.fori_loop(0, ZCH, out_chunk, 0)

    pltpu.sync_copy(iota_hbm.at[sid], idxz_v)
    pltpu.sync_copy(rowc_hbm.at[pl.ds(cb, CH_PER_TILE)], rowi_v)

    fill_rows(z16)
    zero_acc()
    plsc.subcore_barrier()

    HALF = CH_PER_TILE // 2
    NPAIR = HALF // 2

    for hp in range(2):
        hc = hp * HALF
        pltpu.sync_copy(colc_hbm.at[pl.ds(cb + hc, HALF)], coli_v)
        pltpu.async_copy(x_hbm.at[coli_v.at[0]], rows_a, gsem_a)
        pltpu.async_copy(x_hbm.at[coli_v.at[1]], rows_b, gsem_b)

        def pair(t, _):
            pltpu.make_async_copy(x_hbm.at[coli_v.at[2 * t]], rows_a,
                                  gsem_a).wait()
            pltpu.sync_copy(rows_a, agg_sh.at[rowi_v.at[hc + 2 * t]],
                            add=True)

            @pl.when(t < NPAIR - 1)
            def _():
                pltpu.async_copy(x_hbm.at[coli_v.at[2 * t + 2]], rows_a,
                                 gsem_a)

            pltpu.make_async_copy(x_hbm.at[coli_v.at[2 * t + 1]], rows_b,
                                  gsem_b).wait()
            pltpu.sync_copy(rows_b, agg_sh.at[rowi_v.at[hc + 2 * t + 1]],
                            add=True)

            @pl.when(t < NPAIR - 1)
            def _():
                pltpu.async_copy(x_hbm.at[coli_v.at[2 * t + 3]], rows_b,
                                 gsem_b)

            return 0

        lax.fori_loop(0, NPAIR, pair, 0)

    plsc.subcore_barrier()
    copy_out(agg_hbm)
    plsc.subcore_barrier()

    fill_rows(o16)

    def batch_deg(bi, _):
        for j in range(IDXB):
            pltpu.async_copy(rows_a, agg_sh.at[rowi_v.at[bi * IDXB + j]],
                             dsem, add=True)
        for j in range(IDXB):
            pltpu.make_async_copy(rows_a,
                                  agg_sh.at[rowi_v.at[bi * IDXB + j]],
                                  dsem).wait()
        return 0

    lax.fori_loop(0, CH_PER_TILE // IDXB, batch_deg, 0)
    plsc.subcore_barrier()
    copy_out(deg_hbm)


@functools.cache
def _sc_scatter_kernel():
    return pl.kernel(
        _sc_body,
        out_type=(
            jax.ShapeDtypeStruct((NC * NPAD, D), jnp.float32),
            jax.ShapeDtypeStruct((NC * NPAD, D), jnp.float32),
        ),
        mesh=plsc.VectorSubcoreMesh(core_axis_name="c", subcore_axis_name="s",
                                    num_cores=NC, num_subcores=NS),
        scratch_types=[
            pltpu.MemorySpace.VMEM_SHARED((NPAD, D), jnp.float32),
            pltpu.VMEM((CH_PER_TILE, CHUNK), jnp.int32),
            pltpu.VMEM((CH_PER_TILE // 2, CHUNK), jnp.int32),
            pltpu.VMEM((ZCH, CHUNK), jnp.int32),
            pltpu.VMEM((CHUNK, D), jnp.float32),
            pltpu.VMEM((CHUNK, D), jnp.float32),
            pltpu.SemaphoreType.DMA,
            pltpu.SemaphoreType.DMA,
            pltpu.SemaphoreType.DMA,
        ],
    )


ROWB = 1024
NBLK = NPAD // ROWB


def _lin_body(x_ref, agg_ref, deg_ref, w_ref, b_ref,
              h_ref, s_ref, q_ref, sacc, qacc):
    i = pl.program_id(0)
    agg = agg_ref[0] + agg_ref[1]
    valid = (lax.broadcasted_iota(jnp.int32, (ROWB, 1), 0) + i * ROWB
             < N_NODES).astype(jnp.float32)
    deg = (deg_ref[0, :, 0:1] - agg_ref[0, :, 0:1]
           + deg_ref[1, :, 0:1] - agg_ref[1, :, 0:1])
    scale = 1.0 / jnp.maximum(deg, 1.0)
    hin = x_ref[...] + agg * scale
    h = lax.dot_general(hin, w_ref[...], (((1,), (1,)), ((), ())),
                        preferred_element_type=jnp.float32) + b_ref[...]
    h_ref[...] = h

    @pl.when(i == 0)
    def _():
        sacc[...] = jnp.zeros_like(sacc)
        qacc[...] = jnp.zeros_like(qacc)

    hm = h * valid
    sacc[...] += jnp.sum(hm, axis=0, keepdims=True)
    qacc[...] += jnp.sum(hm * hm, axis=0, keepdims=True)

    @pl.when(i == NBLK - 1)
    def _():
        s_ref[...] = sacc[...]
        q_ref[...] = qacc[...]


def _norm_body(h_ref, s_ref, q_ref, g_ref, be_ref, o_ref):
    inv_n = 1.0 / N_NODES
    mean = s_ref[...] * inv_n
    var = q_ref[...] * inv_n - mean * mean
    rstd = lax.rsqrt(var + EPS)
    o_ref[...] = jnp.maximum(
        (h_ref[...] - mean) * rstd * g_ref[...] + be_ref[...], 0.0)


def kernel(x, edge_index, W, b, gamma, beta):
    row = edge_index[0]
    col = edge_index[1]
    pad = E_PAD - N_EDGES
    row_p = jnp.concatenate([row, jnp.full((pad,), N_NODES, jnp.int32)])
    col_p = jnp.concatenate([col, jnp.zeros((pad,), jnp.int32)])
    rowc = row_p.reshape(NW * CH_PER_TILE, CHUNK)
    colc = col_p.reshape(NW * CH_PER_TILE, CHUNK)

    iota = jnp.arange(NPAD, dtype=jnp.int32).reshape(NS, ZCH, CHUNK)

    agg_flat, deg_flat = _sc_scatter_kernel()(x, rowc, colc, iota)
    agg3 = agg_flat.reshape(NC, NPAD, D)
    deg3 = deg_flat.reshape(NC, NPAD, D)
    x_p = jnp.concatenate(
        [x, jnp.zeros((NPAD - N_NODES, D), jnp.float32)], axis=0)

    b2 = b.reshape(1, D)
    g2 = gamma.reshape(1, D)
    be2 = beta.reshape(1, D)

    h, ssum, ssq = pl.pallas_call(
        _lin_body,
        grid=(NBLK,),
        in_specs=[
            pl.BlockSpec((ROWB, D), lambda i: (i, 0)),
            pl.BlockSpec((NC, ROWB, D), lambda i: (0, i, 0)),
            pl.BlockSpec((NC, ROWB, D), lambda i: (0, i, 0)),
            pl.BlockSpec((D, D), lambda i: (0, 0)),
            pl.BlockSpec((1, D), lambda i: (0, 0)),
        ],
        out_specs=[
            pl.BlockSpec((ROWB, D), lambda i: (i, 0)),
            pl.BlockSpec((1, D), lambda i: (0, 0)),
            pl.BlockSpec((1, D), lambda i: (0, 0)),
        ],
        out_shape=[
            jax.ShapeDtypeStruct((NPAD, D), jnp.float32),
            jax.ShapeDtypeStruct((1, D), jnp.float32),
            jax.ShapeDtypeStruct((1, D), jnp.float32),
        ],
        scratch_shapes=[
            pltpu.VMEM((1, D), jnp.float32),
            pltpu.VMEM((1, D), jnp.float32),
        ],
    )(x_p, agg3, deg3, W, b2)

    out = pl.pallas_call(
        _norm_body,
        grid=(NBLK,),
        in_specs=[
            pl.BlockSpec((ROWB, D), lambda i: (i, 0)),
            pl.BlockSpec((1, D), lambda i: (0, 0)),
            pl.BlockSpec((1, D), lambda i: (0, 0)),
            pl.BlockSpec((1, D), lambda i: (0, 0)),
            pl.BlockSpec((1, D), lambda i: (0, 0)),
        ],
        out_specs=pl.BlockSpec((ROWB, D), lambda i: (i, 0)),
        out_shape=jax.ShapeDtypeStruct((NPAD, D), jnp.float32),
    )(h, ssum, ssq, g2, be2)

    return out[:N_NODES]
